# Initial kernel scaffold; baseline (speedup 1.0000x reference)
import jax
import jax.numpy as jnp
from jax.experimental import pallas as pl


def kernel(x, batch, Ws, bs, Wh, bh, Wout1, Wout2, bout2):
    raise NotImplementedError("write your pallas kernel here")


# speedup vs baseline: 4.5235x; 4.5235x over previous
import jax
import jax.numpy as jnp
from jax import lax
from jax.experimental import pallas as pl
from jax.experimental.pallas import tpu as pltpu
from jax.experimental.pallas import tpu_sc as plsc

N = 10000
K = 10
NP = 10240
BI = 256
BJ = 512
L = 16
NC = 2
NS = 16
NW = NC * NS
NG = N // L
GPW = (NG + NW - 1) // NW
SLAB = 2048
NSLAB = 6
MASKD = 1e10


def _d_body(si, sj, sqi, sqj, bi, bj, out):
    cross = lax.dot_general(si[...], sj[...], (((1,), (1,)), ((), ())))
    d = sqi[...] + sqj[...] - 2.0 * cross
    d = jnp.maximum(d, 0.0)
    out[...] = jnp.where(bi[...] != bj[...], MASKD, d)


_d_kernel = pl.pallas_call(
    _d_body,
    out_shape=jax.ShapeDtypeStruct((NP, NP), jnp.float32),
    grid=(NP // BI, NP // BJ),
    in_specs=[
        pl.BlockSpec((BI, 2), lambda i, j: (i, 0)),
        pl.BlockSpec((BJ, 2), lambda i, j: (j, 0)),
        pl.BlockSpec((BI, 1), lambda i, j: (i, 0)),
        pl.BlockSpec((1, BJ), lambda i, j: (0, j)),
        pl.BlockSpec((BI, 1), lambda i, j: (i, 0)),
        pl.BlockSpec((1, BJ), lambda i, j: (0, j)),
    ],
    out_specs=pl.BlockSpec((BI, BJ), lambda i, j: (i, j)),
)


def _sc_body(d_h, h0_h, h1_h, h2_h, x0_h, x1_h, x2_h, lo_h, hi_h, w_h,
             out_h,
             h0, h1, h2, x0, x1, x2, lov, hiv, wv, dtile, otmp):
    cid = lax.axis_index("c")
    sid = lax.axis_index("s")
    wid = sid * NC + cid

    pltpu.sync_copy(h0_h, h0)
    pltpu.sync_copy(h1_h, h1)
    pltpu.sync_copy(h2_h, h2)
    pltpu.sync_copy(x0_h, x0)
    pltpu.sync_copy(x1_h, x1)
    pltpu.sync_copy(x2_h, x2)
    pltpu.sync_copy(lo_h, lov)
    pltpu.sync_copy(hi_h, hiv)
    pltpu.sync_copy(w_h, wv)

    def wrow(k):
        return wv[pl.ds(k * L, L)]

    wo1 = [wrow(i) for i in range(3)]
    wo2 = [wrow(3 + i) for i in range(6)]
    wb2 = wrow(9)
    rows = lax.iota(jnp.int32, L)

    def group_body(gi, _):
        g = wid + gi * NW

        @pl.when(g < NG)
        def _():
            base = g * L
            gv = jnp.full((L,), g, jnp.int32)
            lo = jnp.min(plsc.load_gather(lov, [gv]))
            hi = jnp.min(plsc.load_gather(hiv, [gv]))
            A = jnp.bitwise_and(lo, -128)

            init = tuple([jnp.full((L,), MASKD, jnp.float32)] * K
                         + [jnp.zeros((L,), jnp.int32)] * K)

            def slab_body(t, carry):
                s_un = A + t * SLAB
                sc = pl.multiple_of(jnp.minimum(s_un, NP - SLAB), 128)
                r_lo = jnp.maximum(lo, s_un)
                r_hi = jnp.minimum(hi, s_un + SLAB)

                def do(cr):
                    pltpu.sync_copy(
                        d_h.at[pl.ds(base, L), pl.ds(sc, SLAB)], dtile)

                    def cand(c, cr2):
                        col = jnp.full((L,), c, jnp.int32)
                        dcol = plsc.load_gather(dtile, [rows, col])
                        jv = col + sc

                        def insert(cr3):
                            cr3 = list(cr3)
                            v, iv = dcol, jv
                            for r in range(K):
                                tr, ir = cr3[r], cr3[K + r]
                                ltm = v < tr
                                cr3[r] = jnp.where(ltm, v, tr)
                                cr3[K + r] = jnp.where(ltm, iv, ir)
                                v = jnp.where(ltm, tr, v)
                                iv = jnp.where(ltm, ir, iv)
                            return tuple(cr3)

                        return lax.cond(jnp.any(dcol < cr2[K - 1]), insert,
                                        lambda c3: c3, cr2)

                    return lax.fori_loop(r_lo - sc, r_hi - sc, cand, cr)

                return lax.cond(r_lo < r_hi, do, lambda cr: cr, carry)

            res = lax.fori_loop(0, NSLAB, slab_body, init)

            sm0 = sm1 = sm2 = jnp.zeros((L,), jnp.float32)
            mx0 = mx1 = mx2 = None
            for r in range(K):
                wgt = jnp.exp(-10.0 * res[r])
                ir = res[K + r]
                m0 = plsc.load_gather(h0, [ir]) * wgt
                m1 = plsc.load_gather(h1, [ir]) * wgt
                m2 = plsc.load_gather(h2, [ir]) * wgt
                sm0, sm1, sm2 = sm0 + m0, sm1 + m1, sm2 + m2
                if r == 0:
                    mx0, mx1, mx2 = m0, m1, m2
                else:
                    mx0 = jnp.maximum(mx0, m0)
                    mx1 = jnp.maximum(mx1, m1)
                    mx2 = jnp.maximum(mx2, m2)

            dsl = pl.ds(base, L)
            inv_k = 1.0 / K
            o = (x0[dsl] * wo1[0] + x1[dsl] * wo1[1] + x2[dsl] * wo1[2]
                 + sm0 * inv_k * wo2[0] + sm1 * inv_k * wo2[1]
                 + sm2 * inv_k * wo2[2]
                 + mx0 * wo2[3] + mx1 * wo2[4] + mx2 * wo2[5] + wb2)
            otmp[...] = o
            pltpu.sync_copy(otmp, out_h.at[dsl])

        return 0

    lax.fori_loop(0, GPW, group_body, 0)


_sc_kernel = pl.kernel(
    _sc_body,
    out_type=jax.ShapeDtypeStruct((N,), jnp.float32),
    mesh=plsc.VectorSubcoreMesh(core_axis_name="c", subcore_axis_name="s"),
    compiler_params=pltpu.CompilerParams(needs_layout_passes=False),
    scratch_types=[
        pltpu.VMEM((N,), jnp.float32),
        pltpu.VMEM((N,), jnp.float32),
        pltpu.VMEM((N,), jnp.float32),
        pltpu.VMEM((N,), jnp.float32),
        pltpu.VMEM((N,), jnp.float32),
        pltpu.VMEM((N,), jnp.float32),
        pltpu.VMEM((NG,), jnp.int32),
        pltpu.VMEM((NG,), jnp.int32),
        pltpu.VMEM((10 * L,), jnp.float32),
        pltpu.VMEM((L, SLAB), jnp.float32),
        pltpu.VMEM((L,), jnp.float32),
    ],
)


@jax.jit
def kernel(x, batch, Ws, bs, Wh, bh, Wout1, Wout2, bout2):
    b32 = batch.astype(jnp.int32)
    s = x @ Ws.T + bs
    h = x @ Wh.T + bh
    sq = jnp.sum(s * s, axis=1)

    pad = NP - N
    sp = jnp.pad(s, ((0, pad), (0, 0)))
    sqp = jnp.pad(sq, (0, pad))
    bp = jnp.pad(b32, (0, pad), constant_values=-1)
    d = _d_kernel(sp, sp, sqp[:, None], sqp[None, :], bp[:, None],
                  bp[None, :])

    br = b32.reshape(NG, L)
    lo_arr = jnp.searchsorted(b32, br.min(axis=1), side="left")
    hi_arr = jnp.searchsorted(b32, br.max(axis=1), side="right")
    wsc = jnp.concatenate([Wout1.ravel(), Wout2.ravel(), bout2])
    wvec = jnp.repeat(wsc, L)

    out = _sc_kernel(d, h[:, 0], h[:, 1], h[:, 2],
                     x[:, 0], x[:, 1], x[:, 2],
                     lo_arr.astype(jnp.int32), hi_arr.astype(jnp.int32),
                     wvec)
    return out[:, None]


# speedup vs baseline: 4.6502x; 1.0280x over previous
import jax
import jax.numpy as jnp
from jax import lax
from jax.experimental import pallas as pl
from jax.experimental.pallas import tpu as pltpu
from jax.experimental.pallas import tpu_sc as plsc

N = 10000
K = 10
NP = 10240
BI = 256
BJ = 512
L = 16
NC = 2
NS = 16
NW = NC * NS
NG = N // L
GPW = (NG + NW - 1) // NW
SLAB = 2048
NSLAB = 6
MASKD = 1e10


def _d_body(rmin, rmax, cmin, cmax, si, sj, sqi, sqj, bi, bj, out):
    i = pl.program_id(0)
    j = pl.program_id(1)
    overlap = jnp.logical_and(rmax[i] >= cmin[j], rmin[i] <= cmax[j])

    @pl.when(overlap)
    def _():
        cross = lax.dot_general(si[...], sj[...], (((1,), (1,)), ((), ())))
        d = sqi[...] + sqj[...] - 2.0 * cross
        d = jnp.maximum(d, 0.0)
        out[...] = jnp.where(bi[...] != bj[...], MASKD, d)


_d_kernel = pl.pallas_call(
    _d_body,
    out_shape=jax.ShapeDtypeStruct((NP, NP), jnp.float32),
    grid=(NP // BI, NP // BJ),
    in_specs=[
        pl.BlockSpec(memory_space=pltpu.SMEM),
        pl.BlockSpec(memory_space=pltpu.SMEM),
        pl.BlockSpec(memory_space=pltpu.SMEM),
        pl.BlockSpec(memory_space=pltpu.SMEM),
        pl.BlockSpec((BI, 2), lambda i, j: (i, 0)),
        pl.BlockSpec((BJ, 2), lambda i, j: (j, 0)),
        pl.BlockSpec((BI, 1), lambda i, j: (i, 0)),
        pl.BlockSpec((1, BJ), lambda i, j: (0, j)),
        pl.BlockSpec((BI, 1), lambda i, j: (i, 0)),
        pl.BlockSpec((1, BJ), lambda i, j: (0, j)),
    ],
    out_specs=pl.BlockSpec((BI, BJ), lambda i, j: (i, j)),
)


def _sc_body(d_h, h0_h, h1_h, h2_h, x0_h, x1_h, x2_h, lo_h, hi_h, w_h,
             out_h,
             h0, h1, h2, x0, x1, x2, lov, hiv, wv, dtile, otmp):
    cid = lax.axis_index("c")
    sid = lax.axis_index("s")
    wid = sid * NC + cid

    pltpu.sync_copy(h0_h, h0)
    pltpu.sync_copy(h1_h, h1)
    pltpu.sync_copy(h2_h, h2)
    pltpu.sync_copy(x0_h, x0)
    pltpu.sync_copy(x1_h, x1)
    pltpu.sync_copy(x2_h, x2)
    pltpu.sync_copy(lo_h, lov)
    pltpu.sync_copy(hi_h, hiv)
    pltpu.sync_copy(w_h, wv)

    def wrow(k):
        return wv[pl.ds(k * L, L)]

    wo1 = [wrow(i) for i in range(3)]
    wo2 = [wrow(3 + i) for i in range(6)]
    wb2 = wrow(9)
    rows = lax.iota(jnp.int32, L)

    def group_body(gi, _):
        g = wid + gi * NW

        @pl.when(g < NG)
        def _():
            base = g * L
            gv = jnp.full((L,), g, jnp.int32)
            lo = jnp.min(plsc.load_gather(lov, [gv]))
            hi = jnp.min(plsc.load_gather(hiv, [gv]))
            A = jnp.bitwise_and(lo, -128)

            init = tuple([jnp.full((L,), MASKD, jnp.float32)] * K
                         + [jnp.zeros((L,), jnp.int32)] * K)

            def slab_body(t, carry):
                s_un = A + t * SLAB
                sc = pl.multiple_of(jnp.minimum(s_un, NP - SLAB), 128)
                r_lo = jnp.maximum(lo, s_un)
                r_hi = jnp.minimum(hi, s_un + SLAB)

                def do(cr):
                    pltpu.sync_copy(
                        d_h.at[pl.ds(base, L), pl.ds(sc, SLAB)], dtile)

                    def cand(c, cr2):
                        col = jnp.full((L,), c, jnp.int32)
                        dcol = plsc.load_gather(dtile, [rows, col])
                        jv = col + sc

                        def insert(cr3):
                            cr3 = list(cr3)
                            v, iv = dcol, jv
                            for r in range(K):
                                tr, ir = cr3[r], cr3[K + r]
                                ltm = v < tr
                                cr3[r] = jnp.where(ltm, v, tr)
                                cr3[K + r] = jnp.where(ltm, iv, ir)
                                v = jnp.where(ltm, tr, v)
                                iv = jnp.where(ltm, ir, iv)
                            return tuple(cr3)

                        return lax.cond(jnp.any(dcol < cr2[K - 1]), insert,
                                        lambda c3: c3, cr2)

                    return lax.fori_loop(r_lo - sc, r_hi - sc, cand, cr)

                return lax.cond(r_lo < r_hi, do, lambda cr: cr, carry)

            res = lax.fori_loop(0, NSLAB, slab_body, init)

            sm0 = sm1 = sm2 = jnp.zeros((L,), jnp.float32)
            mx0 = mx1 = mx2 = None
            for r in range(K):
                wgt = jnp.exp(-10.0 * res[r])
                ir = res[K + r]
                m0 = plsc.load_gather(h0, [ir]) * wgt
                m1 = plsc.load_gather(h1, [ir]) * wgt
                m2 = plsc.load_gather(h2, [ir]) * wgt
                sm0, sm1, sm2 = sm0 + m0, sm1 + m1, sm2 + m2
                if r == 0:
                    mx0, mx1, mx2 = m0, m1, m2
                else:
                    mx0 = jnp.maximum(mx0, m0)
                    mx1 = jnp.maximum(mx1, m1)
                    mx2 = jnp.maximum(mx2, m2)

            dsl = pl.ds(base, L)
            inv_k = 1.0 / K
            o = (x0[dsl] * wo1[0] + x1[dsl] * wo1[1] + x2[dsl] * wo1[2]
                 + sm0 * inv_k * wo2[0] + sm1 * inv_k * wo2[1]
                 + sm2 * inv_k * wo2[2]
                 + mx0 * wo2[3] + mx1 * wo2[4] + mx2 * wo2[5] + wb2)
            otmp[...] = o
            pltpu.sync_copy(otmp, out_h.at[dsl])

        return 0

    lax.fori_loop(0, GPW, group_body, 0)


_sc_kernel = pl.kernel(
    _sc_body,
    out_type=jax.ShapeDtypeStruct((N,), jnp.float32),
    mesh=plsc.VectorSubcoreMesh(core_axis_name="c", subcore_axis_name="s"),
    compiler_params=pltpu.CompilerParams(needs_layout_passes=False),
    scratch_types=[
        pltpu.VMEM((N,), jnp.float32),
        pltpu.VMEM((N,), jnp.float32),
        pltpu.VMEM((N,), jnp.float32),
        pltpu.VMEM((N,), jnp.float32),
        pltpu.VMEM((N,), jnp.float32),
        pltpu.VMEM((N,), jnp.float32),
        pltpu.VMEM((NG,), jnp.int32),
        pltpu.VMEM((NG,), jnp.int32),
        pltpu.VMEM((10 * L,), jnp.float32),
        pltpu.VMEM((L, SLAB), jnp.float32),
        pltpu.VMEM((L,), jnp.float32),
    ],
)


@jax.jit
def kernel(x, batch, Ws, bs, Wh, bh, Wout1, Wout2, bout2):
    b32 = batch.astype(jnp.int32)
    s = x @ Ws.T + bs
    h = x @ Wh.T + bh
    sq = jnp.sum(s * s, axis=1)

    pad = NP - N
    sp = jnp.pad(s, ((0, pad), (0, 0)))
    sqp = jnp.pad(sq, (0, pad))
    bp = jnp.pad(b32, (0, pad), constant_values=-1)
    rmin = bp.reshape(NP // BI, BI).min(axis=1)
    rmax = bp.reshape(NP // BI, BI).max(axis=1)
    cmin = bp.reshape(NP // BJ, BJ).min(axis=1)
    cmax = bp.reshape(NP // BJ, BJ).max(axis=1)
    d = _d_kernel(rmin, rmax, cmin, cmax,
                  sp, sp, sqp[:, None], sqp[None, :], bp[:, None],
                  bp[None, :])

    br = b32.reshape(NG, L)
    lo_arr = jnp.searchsorted(b32, br.min(axis=1), side="left")
    hi_arr = jnp.searchsorted(b32, br.max(axis=1), side="right")
    wsc = jnp.concatenate([Wout1.ravel(), Wout2.ravel(), bout2])
    wvec = jnp.repeat(wsc, L)

    out = _sc_kernel(d, h[:, 0], h[:, 1], h[:, 2],
                     x[:, 0], x[:, 1], x[:, 2],
                     lo_arr.astype(jnp.int32), hi_arr.astype(jnp.int32),
                     wvec)
    return out[:, None]
